# Initial kernel scaffold; baseline (speedup 1.0000x reference)
import jax
import jax.numpy as jnp
from jax.experimental import pallas as pl


def kernel(x, edge_index, edge_weight, batch, params):
    raise NotImplementedError("write your pallas kernel here")


# speedup vs baseline: 4.6849x; 4.6849x over previous
import functools

import jax
import jax.numpy as jnp
from jax import lax
from jax.experimental import pallas as pl
from jax.experimental.pallas import tpu as pltpu
from jax.experimental.pallas import tpu_sc as plsc

F32 = jnp.float32
N = 50000
G = 1024
NP = 53248
EP = 819200
NC, NS, LANES = 2, 16, 16
CH = 128
SB = 8
ROWS_PER_TILE = NP // NS
ZR = 416
PG = 1152


def _mesh():
    return plsc.VectorSubcoreMesh(core_axis_name="c", subcore_axis_name="s")


def _sc_params():
    return pltpu.CompilerParams(use_tc_tiling_on_sc=False)


def _zero_zbuf(zbuf, db):
    @pl.loop(0, zbuf.shape[0])
    def _(i):
        for k in range(db // LANES):
            zbuf[i, pl.ds(k * LANES, LANES)] = jnp.zeros((LANES,), F32)


def _zero_acc(acc, zbuf, sid, nrows_per_tile, zr):
    nz = nrows_per_tile // zr

    @pl.loop(0, nz)
    def _(j):
        pltpu.sync_copy(zbuf, acc.at[pl.ds(sid * nrows_per_tile + j * zr, zr)])


def _edge_scan(h_hbm, src_hbm, dst_hbm, ew_hbm, acc, sidx, didx, eww, rows,
               sem, chunk_base, nouter, db):
    @pl.loop(0, nouter)
    def _(o):
        cb = chunk_base + o * SB
        pltpu.sync_copy(src_hbm.at[pl.ds(cb, SB)], sidx)
        pltpu.sync_copy(dst_hbm.at[pl.ds(cb, SB)], didx)
        pltpu.sync_copy(ew_hbm.at[pl.ds(cb, SB)], eww)

        @pl.loop(0, SB)
        def _(j):
            pltpu.async_copy(h_hbm.at[sidx.at[j]], rows, sem).wait()

            @pl.loop(0, CH // LANES)
            def _(gidx):
                wv = eww[j, pl.ds(gidx * LANES, LANES)]
                for l in range(LANES):
                    w = wv[l]
                    e = gidx * LANES + l
                    for k in range(db // LANES):
                        sl = (e, pl.ds(k * LANES, LANES))
                        rows[sl] = rows[sl] * w

            pltpu.sync_copy(rows, acc.at[didx.at[j]], add=True)


def _scatter_layer_split_edges(db, h, srcb, dstb, ewb):
    chunks_per_core = EP // 2 // CH
    chunks_per_tile = chunks_per_core // NS
    nouter = chunks_per_tile // SB

    @functools.partial(
        pl.kernel,
        out_type=(jax.ShapeDtypeStruct((NP, db), F32),) * 2,
        mesh=_mesh(),
        compiler_params=_sc_params(),
        scratch_types=[
            pltpu.VMEM_SHARED((NP, db), F32),
            pltpu.VMEM((SB, CH), jnp.int32),
            pltpu.VMEM((SB, CH), jnp.int32),
            pltpu.VMEM((SB, CH), F32),
            pltpu.VMEM((CH, db), F32),
            pltpu.VMEM((ZR, db), F32),
            pltpu.SemaphoreType.DMA,
        ],
    )
    def k(h_hbm, src_hbm, dst_hbm, ew_hbm, o0, o1, acc, sidx, didx, eww,
          rows, zbuf, sem):
        cid = lax.axis_index("c")
        sid = lax.axis_index("s")
        _zero_zbuf(zbuf, db)
        _zero_acc(acc, zbuf, sid, ROWS_PER_TILE, ZR)
        plsc.subcore_barrier()
        chunk_base = cid * chunks_per_core + sid * chunks_per_tile
        _edge_scan(h_hbm, src_hbm, dst_hbm, ew_hbm, acc, sidx, didx, eww,
                   rows, sem, chunk_base, nouter, db)
        plsc.subcore_barrier()
        sl = pl.ds(sid * ROWS_PER_TILE, ROWS_PER_TILE)

        @pl.when(cid == 0)
        def _():
            pltpu.sync_copy(acc.at[sl], o0.at[sl])

        @pl.when(cid == 1)
        def _():
            pltpu.sync_copy(acc.at[sl], o1.at[sl])

    return k(h, srcb, dstb, ewb)


def _scatter_layer_split_cols(h_blocks, srcb, dstb, ewb):
    K = len(h_blocks)
    db = 32
    chunks_per_tile = EP // CH // NS
    nouter = chunks_per_tile // SB

    @functools.partial(
        pl.kernel,
        out_type=(jax.ShapeDtypeStruct((NP, db), F32),) * K,
        mesh=_mesh(),
        compiler_params=_sc_params(),
        scratch_types=[
            pltpu.VMEM_SHARED((NP, db), F32),
            pltpu.VMEM((SB, CH), jnp.int32),
            pltpu.VMEM((SB, CH), jnp.int32),
            pltpu.VMEM((SB, CH), F32),
            pltpu.VMEM((CH, db), F32),
            pltpu.VMEM((ZR, db), F32),
            pltpu.SemaphoreType.DMA,
        ],
    )
    def k(*refs):
        h_hbms = refs[:K]
        src_hbm, dst_hbm, ew_hbm = refs[K:K + 3]
        outs = refs[K + 3:2 * K + 3]
        acc, sidx, didx, eww, rows, zbuf, sem = refs[2 * K + 3:]
        cid = lax.axis_index("c")
        sid = lax.axis_index("s")
        _zero_zbuf(zbuf, db)
        for blk in range(K):
            @pl.when(cid == (blk % 2))
            def _(blk=blk):
                _zero_acc(acc, zbuf, sid, ROWS_PER_TILE, ZR)
                plsc.subcore_barrier()
                _edge_scan(h_hbms[blk], src_hbm, dst_hbm, ew_hbm, acc, sidx,
                           didx, eww, rows, sem, sid * chunks_per_tile,
                           nouter, db)
                plsc.subcore_barrier()
                sl = pl.ds(sid * ROWS_PER_TILE, ROWS_PER_TILE)
                pltpu.sync_copy(acc.at[sl], outs[blk].at[sl])
                plsc.subcore_barrier()

    return k(*h_blocks, srcb, dstb, ewb)


def _pool_kernel(y_blocks, batchb):
    K = len(y_blocks)
    db = 32
    chunks_per_tile = NP // CH // (NC * NS)
    prt = PG // NS

    @functools.partial(
        pl.kernel,
        out_type=tuple([jax.ShapeDtypeStruct((PG, db), F32)] * (2 * K)
                       + [jax.ShapeDtypeStruct((PG, LANES), F32)] * 2),
        mesh=_mesh(),
        compiler_params=_sc_params(),
        scratch_types=[pltpu.VMEM_SHARED((PG, db), F32)] * K + [
            pltpu.VMEM_SHARED((PG, LANES), F32),
            pltpu.VMEM((chunks_per_tile, CH), jnp.int32),
            pltpu.VMEM((CH, db), F32),
            pltpu.VMEM((CH, LANES), F32),
            pltpu.VMEM((prt, db), F32),
            pltpu.VMEM((prt, LANES), F32),
            pltpu.SemaphoreType.DMA,
        ],
    )
    def k(*refs):
        y_hbms = refs[:K]
        batch_hbm = refs[K]
        pouts = refs[K + 1:K + 1 + 2 * K]
        couts = refs[K + 1 + 2 * K:K + 3 + 2 * K]
        accs = refs[K + 3 + 2 * K:2 * K + 3 + 2 * K]
        cacc, bidx, rows, ones, zbuf, zbuf16, sem = refs[2 * K + 3 + 2 * K:]
        cid = lax.axis_index("c")
        sid = lax.axis_index("s")
        wid = cid * NS + sid
        _zero_zbuf(zbuf, db)
        _zero_zbuf(zbuf16, LANES)
        for a in accs:
            _zero_acc(a, zbuf, sid, prt, prt)
        _zero_acc(cacc, zbuf16, sid, prt, prt)

        @pl.loop(0, CH)
        def _(i):
            ones[i, pl.ds(0, LANES)] = jnp.ones((LANES,), F32)

        pltpu.sync_copy(batch_hbm.at[wid], bidx)
        plsc.subcore_barrier()

        @pl.loop(0, chunks_per_tile)
        def _(j):
            cb = wid * chunks_per_tile + j
            for blk in range(K):
                pltpu.sync_copy(y_hbms[blk].at[pl.ds(cb * CH, CH)], rows)
                pltpu.sync_copy(rows, accs[blk].at[bidx.at[j]], add=True)
            pltpu.sync_copy(ones, cacc.at[bidx.at[j]], add=True)

        plsc.subcore_barrier()
        sl = pl.ds(sid * prt, prt)

        @pl.when(cid == 0)
        def _():
            for blk in range(K):
                pltpu.sync_copy(accs[blk].at[sl], pouts[blk].at[sl])
            pltpu.sync_copy(cacc.at[sl], couts[0].at[sl])

        @pl.when(cid == 1)
        def _():
            for blk in range(K):
                pltpu.sync_copy(accs[blk].at[sl], pouts[K + blk].at[sl])
            pltpu.sync_copy(cacc.at[sl], couts[1].at[sl])

    return k(*y_blocks, batchb)


def _first_matmul(x_pad, w1p):
    def body(xr, wr, orf):
        orf[...] = jnp.dot(xr[...], wr[...], preferred_element_type=F32)

    return pl.pallas_call(
        body,
        grid=(NP // 1024,),
        in_specs=[
            pl.BlockSpec((1024, 128), lambda i: (i, 0)),
            pl.BlockSpec((128, 16), lambda i: (0, 0)),
        ],
        out_specs=pl.BlockSpec((1024, 16), lambda i: (i, 0)),
        out_shape=jax.ShapeDtypeStruct((NP, 16), F32),
    )(x_pad, w1p)


def _stats(parts):
    db = parts[0].shape[1]
    nparts = len(parts)

    def body(*refs):
        zb = refs[0][...]
        for r in refs[1:nparts]:
            zb = zb + r[...]
        orf = refs[nparts]
        s = jnp.sum(zb, axis=0)[None, :]
        q = jnp.sum(zb * zb, axis=0)[None, :]
        upd = jnp.concatenate([s, q, jnp.zeros((6, db), F32)], axis=0)

        @pl.when(pl.program_id(0) == 0)
        def _():
            orf[...] = jnp.zeros_like(orf)

        orf[...] += upd

    return pl.pallas_call(
        body,
        grid=(N // 1000,),
        in_specs=[pl.BlockSpec((1000, db), lambda i: (i, 0))] * nparts,
        out_specs=pl.BlockSpec((8, db), lambda i: (0, 0)),
        out_shape=jax.ShapeDtypeStruct((8, db), F32),
    )(*parts)


def _apply(part_groups, stats_list, g_list, be_list, wn_blocks=None,
           out_db=16):
    B = len(part_groups)
    db = part_groups[0][0].shape[1]
    npart = len(part_groups[0])
    if wn_blocks is not None:
        dnext = wn_blocks[0].shape[1]
        nout = dnext // out_db
        out_w = out_db
    else:
        nout = B
        out_w = db

    def body(*refs):
        i = 0
        zs = []
        for b in range(B):
            z = refs[i][...]
            for p in range(1, npart):
                z = z + refs[i + p][...]
            i += npart
            zs.append(z)
        sts = [refs[i + b][...] for b in range(B)]
        i += B
        gs = [refs[i + b][...] for b in range(B)]
        i += B
        bes = [refs[i + b][...] for b in range(B)]
        i += B
        wns = None
        if wn_blocks is not None:
            wns = [refs[i + b][...] for b in range(B)]
            i += B
        orefs = refs[i:]
        ys = []
        for b in range(B):
            m = sts[b][0:1, :] / float(N)
            v = sts[b][1:2, :] / float(N) - m * m
            r = lax.rsqrt(v + 1e-5)
            y = (zs[b] - m) * (r * gs[b]) + bes[b]
            ys.append(jnp.where(y >= 0, y, 0.01 * y))
        if wns is None:
            for b in range(B):
                orefs[b][...] = ys[b]
        else:
            acc = jnp.dot(ys[0], wns[0], preferred_element_type=F32)
            for b in range(1, B):
                acc += jnp.dot(ys[b], wns[b], preferred_element_type=F32)
            for j in range(nout):
                orefs[j][...] = acc[:, j * out_db:(j + 1) * out_db]

    in_specs = []
    args = []
    for b in range(B):
        for p in part_groups[b]:
            args.append(p)
            in_specs.append(pl.BlockSpec((1024, db), lambda i: (i, 0)))
    for st in stats_list:
        args.append(st)
        in_specs.append(pl.BlockSpec((8, db), lambda i: (0, 0)))
    for gl in g_list:
        args.append(gl)
        in_specs.append(pl.BlockSpec((1, db), lambda i: (0, 0)))
    for bl in be_list:
        args.append(bl)
        in_specs.append(pl.BlockSpec((1, db), lambda i: (0, 0)))
    if wn_blocks is not None:
        for w in wn_blocks:
            args.append(w)
            in_specs.append(pl.BlockSpec(w.shape, lambda i: (0, 0)))

    return pl.pallas_call(
        body,
        grid=(NP // 1024,),
        in_specs=in_specs,
        out_specs=[pl.BlockSpec((1024, out_w), lambda i: (i, 0))] * nout,
        out_shape=[jax.ShapeDtypeStruct((NP, out_w), F32)] * nout,
    )(*args)


def _fc(pool_parts, cnt_parts, wf1p, bf1p, wf2p, bf2p, wf3p, bf3p):
    K = 4

    def body(*refs):
        ps = refs[:2 * K]
        c0, c1 = refs[2 * K], refs[2 * K + 1]
        w1, b1, w2, b2, w3, b3 = refs[2 * K + 2:2 * K + 8]
        orf = refs[2 * K + 8]
        sums = [ps[b][...] + ps[K + b][...] for b in range(K)]
        s = jnp.concatenate(sums, axis=1)
        cnt = (c0[...] + c1[...])[:, 0:1]
        pooled = s / jnp.maximum(cnt, 1.0)
        a = jnp.dot(pooled, w1[...], preferred_element_type=F32) + b1[...]
        a = jnp.where(a >= 0, a, 0.01 * a)
        a = jnp.dot(a, w2[...], preferred_element_type=F32) + b2[...]
        a = jnp.where(a >= 0, a, 0.01 * a)
        orf[...] = jnp.dot(a, w3[...], preferred_element_type=F32) + b3[...]

    in_specs = (
        [pl.BlockSpec((1024, 32), lambda i: (0, 0))] * (2 * K)
        + [pl.BlockSpec((1024, 16), lambda i: (0, 0))] * 2
        + [pl.BlockSpec((128, 32), lambda i: (0, 0)),
           pl.BlockSpec((1, 32), lambda i: (0, 0)),
           pl.BlockSpec((32, 32), lambda i: (0, 0)),
           pl.BlockSpec((1, 32), lambda i: (0, 0)),
           pl.BlockSpec((32, 128), lambda i: (0, 0)),
           pl.BlockSpec((1, 128), lambda i: (0, 0))]
    )
    return pl.pallas_call(
        body,
        grid=(1,),
        in_specs=in_specs,
        out_specs=pl.BlockSpec((1024, 128), lambda i: (0, 0)),
        out_shape=jax.ShapeDtypeStruct((1024, 128), F32),
    )(*pool_parts, *cnt_parts, wf1p, bf1p, wf2p, bf2p, wf3p, bf3p)


def kernel(x, edge_index, edge_weight, batch, params):
    E = edge_weight.shape[0]
    pad_e = EP - E
    src = edge_index[0].astype(jnp.int32)
    dst = edge_index[1].astype(jnp.int32)
    pad_idx = (jnp.arange(pad_e, dtype=jnp.int32) * 131) % N
    srcb = jnp.concatenate([src, pad_idx]).reshape(EP // CH, CH)
    dstb = jnp.concatenate([dst, pad_idx]).reshape(EP // CH, CH)
    ewb = jnp.concatenate(
        [edge_weight.astype(F32), jnp.zeros((pad_e,), F32)]
    ).reshape(EP // CH, CH)
    batchb = jnp.concatenate(
        [batch.astype(jnp.int32), jnp.full((NP - N,), G, jnp.int32)]
    ).reshape(NC * NS, NP // CH // (NC * NS), CH)

    p = params
    x_pad = jnp.zeros((NP, 128), F32).at[:N, :6].set(x.astype(F32))
    w1p = jnp.zeros((128, 16), F32).at[:6].set(p["W1"])

    def row(a):
        return a.reshape(1, -1)

    h = _first_matmul(x_pad, w1p)
    p0, p1 = _scatter_layer_split_edges(16, h, srcb, dstb, ewb)
    st = _stats([p0, p1])
    (h,) = _apply([[p0, p1]], [st], [row(p["g1"])], [row(p["be1"])],
                  wn_blocks=[p["W2"]], out_db=16)
    p0, p1 = _scatter_layer_split_edges(16, h, srcb, dstb, ewb)
    st = _stats([p0, p1])
    (h,) = _apply([[p0, p1]], [st], [row(p["g2"])], [row(p["be2"])],
                  wn_blocks=[p["W3"]], out_db=32)
    p0, p1 = _scatter_layer_split_edges(32, h, srcb, dstb, ewb)
    st = _stats([p0, p1])
    h4 = _apply([[p0, p1]], [st], [row(p["g3"])], [row(p["be3"])],
                wn_blocks=[p["W4"]], out_db=32)
    z4 = _scatter_layer_split_cols(list(h4), srcb, dstb, ewb)
    st4 = [_stats([z]) for z in z4]
    g4 = [row(p["g4"][:32]), row(p["g4"][32:])]
    be4 = [row(p["be4"][:32]), row(p["be4"][32:])]
    w5b = [p["W5"][:32], p["W5"][32:]]
    h5 = _apply([[z] for z in z4], st4, g4, be4, wn_blocks=w5b, out_db=32)
    z5 = _scatter_layer_split_cols(list(h5), srcb, dstb, ewb)
    st5 = [_stats([z]) for z in z5]
    g5 = [row(p["g5"][32 * b:32 * (b + 1)]) for b in range(4)]
    be5 = [row(p["be5"][32 * b:32 * (b + 1)]) for b in range(4)]
    y5 = _apply([[z] for z in z5], st5, g5, be5, wn_blocks=None)
    pool = _pool_kernel(list(y5), batchb)
    pool_parts, cnt_parts = pool[:8], pool[8:]
    wf1p = jnp.zeros((128, 32), F32).at[:, :30].set(p["Wf1"])
    bf1p = jnp.zeros((1, 32), F32).at[0, :30].set(p["bf1"])
    wf2p = jnp.zeros((32, 32), F32).at[:30, :20].set(p["Wf2"])
    bf2p = jnp.zeros((1, 32), F32).at[0, :20].set(p["bf2"])
    wf3p = jnp.zeros((32, 128), F32).at[:20, :2].set(p["Wf3"])
    bf3p = jnp.zeros((1, 128), F32).at[0, :2].set(p["bf3"])
    out = _fc(pool_parts, cnt_parts, wf1p, bf1p, wf2p, bf2p, wf3p, bf3p)
    return out[:, :2]


# speedup vs baseline: 8.8228x; 1.8832x over previous
import functools

import jax
import jax.numpy as jnp
from jax import lax
from jax.experimental import pallas as pl
from jax.experimental.pallas import tpu as pltpu
from jax.experimental.pallas import tpu_sc as plsc

F32 = jnp.float32
N = 50000
G = 1024
NP = 53248
EP = 819200
NC, NS, LANES = 2, 16, 16
CH = 128
SB = 8
ROWS_PER_TILE = NP // NS
ZR = 416
PG = 1152


def _mesh():
    return plsc.VectorSubcoreMesh(core_axis_name="c", subcore_axis_name="s")


def _sc_params():
    return pltpu.CompilerParams(use_tc_tiling_on_sc=False)


def _zero_zbuf(zbuf, db):
    @pl.loop(0, zbuf.shape[0])
    def _(i):
        for k in range(db // LANES):
            zbuf[i, pl.ds(k * LANES, LANES)] = jnp.zeros((LANES,), F32)


def _zero_rows0(rows, db):
    @pl.loop(0, CH)
    def _(i):
        for k in range(db // LANES):
            rows[0, i, pl.ds(k * LANES, LANES)] = jnp.zeros((LANES,), F32)


def _zero_acc(acc, rows, sid, nrows_per_tile, db):
    nz = nrows_per_tile // CH

    @pl.loop(0, nz)
    def _(j):
        pltpu.sync_copy(rows.at[0],
                        acc.at[pl.ds(sid * nrows_per_tile + j * CH, CH)])


def _edge_scan(h_hbm, src_hbm, dst_hbm, ew_hbm, acc, sidx, didx, eww, rows,
               sem_i, sem_g, sem_s, chunk_base, nblocks, db):
    def stage(bi, h):
        cb = chunk_base + bi * SB
        pltpu.async_copy(src_hbm.at[pl.ds(cb, SB)], sidx.at[h], sem_i)
        pltpu.async_copy(dst_hbm.at[pl.ds(cb, SB)], didx.at[h], sem_i)
        pltpu.async_copy(ew_hbm.at[pl.ds(cb, SB)], eww.at[h], sem_i)

    def wait_stage(bi, h):
        cb = chunk_base + bi * SB
        pltpu.make_async_copy(src_hbm.at[pl.ds(cb, SB)], sidx.at[h],
                              sem_i).wait()
        pltpu.make_async_copy(dst_hbm.at[pl.ds(cb, SB)], didx.at[h],
                              sem_i).wait()
        pltpu.make_async_copy(ew_hbm.at[pl.ds(cb, SB)], eww.at[h],
                              sem_i).wait()

    def gather_start(h, r, b):
        pltpu.async_copy(h_hbm.at[sidx.at[h, r]], rows.at[b], sem_g.at[b])

    def gather_wait(b):
        pltpu.make_async_copy(h_hbm.at[sidx.at[0, 0]], rows.at[b],
                              sem_g.at[b]).wait()

    def scatter_start(b, h, r):
        pltpu.async_copy(rows.at[b], acc.at[didx.at[h, r]], sem_s.at[b],
                         add=True)

    def scatter_wait(b):
        pltpu.make_async_copy(rows.at[b], acc.at[didx.at[0, 0]],
                              sem_s.at[b]).wait()

    def multiply(b, h, r):
        @pl.loop(0, CH // LANES)
        def _(gidx):
            wv = eww[h, r, pl.ds(gidx * LANES, LANES)]
            for l in range(LANES):
                w = wv[l]
                e = gidx * LANES + l
                for k in range(db // LANES):
                    sl = (e, pl.ds(k * LANES, LANES))
                    rows[(b,) + sl] = rows[(b,) + sl] * w

    stage(0, 0)
    wait_stage(0, 0)
    for b in range(2):
        gather_start(0, b, b)

    @pl.loop(0, nblocks)
    def _(bb):
        hb = bb % 2
        hn = (bb + 1) % 2
        for jj in range(SB):
            b = jj % 4
            bg = (jj + 2) % 4
            if jj < 6:
                if jj < 2:
                    @pl.when(bb >= 1)
                    def _(bg=bg):
                        scatter_wait(bg)
                else:
                    scatter_wait(bg)
                gather_start(hb, jj + 2, bg)
                if jj == 3:
                    @pl.when(bb + 1 < nblocks)
                    def _():
                        stage(bb + 1, hn)
            else:
                if jj == 6:
                    @pl.when(bb + 1 < nblocks)
                    def _():
                        wait_stage(bb + 1, hn)

                @pl.when(bb + 1 < nblocks)
                def _(bg=bg, jj=jj):
                    scatter_wait(bg)
                    gather_start(hn, jj - 6, bg)

            gather_wait(b)
            multiply(b, hb, jj)
            scatter_start(b, hb, jj)

    for b in range(4):
        scatter_wait(b)


def _scatter_layer_split_edges(db, h, srcb, dstb, ewb):
    chunks_per_core = EP // 2 // CH
    chunks_per_tile = chunks_per_core // NS
    nouter = chunks_per_tile // SB

    @functools.partial(
        pl.kernel,
        out_type=(jax.ShapeDtypeStruct((NP, db), F32),) * 2,
        mesh=_mesh(),
        compiler_params=_sc_params(),
        scratch_types=[
            pltpu.VMEM_SHARED((NP, db), F32),
            pltpu.VMEM((2, SB, CH), jnp.int32),
            pltpu.VMEM((2, SB, CH), jnp.int32),
            pltpu.VMEM((2, SB, CH), F32),
            pltpu.VMEM((4, CH, db), F32),
            pltpu.SemaphoreType.DMA,
            pltpu.SemaphoreType.DMA((4,)),
            pltpu.SemaphoreType.DMA((4,)),
        ],
    )
    def k(h_hbm, src_hbm, dst_hbm, ew_hbm, o0, o1, acc, sidx, didx, eww,
          rows, sem_i, sem_g, sem_s):
        cid = lax.axis_index("c")
        sid = lax.axis_index("s")
        _zero_rows0(rows, db)
        _zero_acc(acc, rows, sid, ROWS_PER_TILE, db)
        plsc.subcore_barrier()
        chunk_base = cid * chunks_per_core + sid * chunks_per_tile
        _edge_scan(h_hbm, src_hbm, dst_hbm, ew_hbm, acc, sidx, didx, eww,
                   rows, sem_i, sem_g, sem_s, chunk_base, nouter, db)
        plsc.subcore_barrier()
        sl = pl.ds(sid * ROWS_PER_TILE, ROWS_PER_TILE)

        @pl.when(cid == 0)
        def _():
            pltpu.sync_copy(acc.at[sl], o0.at[sl])

        @pl.when(cid == 1)
        def _():
            pltpu.sync_copy(acc.at[sl], o1.at[sl])

    return k(h, srcb, dstb, ewb)


def _scatter_layer_split_cols(h_blocks, srcb, dstb, ewb):
    K = len(h_blocks)
    db = 32
    chunks_per_tile = EP // CH // NS
    nouter = chunks_per_tile // SB

    @functools.partial(
        pl.kernel,
        out_type=(jax.ShapeDtypeStruct((NP, db), F32),) * K,
        mesh=_mesh(),
        compiler_params=_sc_params(),
        scratch_types=[
            pltpu.VMEM_SHARED((NP, db), F32),
            pltpu.VMEM((2, SB, CH), jnp.int32),
            pltpu.VMEM((2, SB, CH), jnp.int32),
            pltpu.VMEM((2, SB, CH), F32),
            pltpu.VMEM((4, CH, db), F32),
            pltpu.SemaphoreType.DMA,
            pltpu.SemaphoreType.DMA((4,)),
            pltpu.SemaphoreType.DMA((4,)),
        ],
    )
    def k(*refs):
        h_hbms = refs[:K]
        src_hbm, dst_hbm, ew_hbm = refs[K:K + 3]
        outs = refs[K + 3:2 * K + 3]
        acc, sidx, didx, eww, rows, sem_i, sem_g, sem_s = refs[2 * K + 3:]
        cid = lax.axis_index("c")
        sid = lax.axis_index("s")
        for blk in range(K):
            @pl.when(cid == (blk % 2))
            def _(blk=blk):
                _zero_rows0(rows, db)
                _zero_acc(acc, rows, sid, ROWS_PER_TILE, db)
                plsc.subcore_barrier()
                _edge_scan(h_hbms[blk], src_hbm, dst_hbm, ew_hbm, acc, sidx,
                           didx, eww, rows, sem_i, sem_g, sem_s,
                           sid * chunks_per_tile, nouter, db)
                plsc.subcore_barrier()
                sl = pl.ds(sid * ROWS_PER_TILE, ROWS_PER_TILE)
                pltpu.sync_copy(acc.at[sl], outs[blk].at[sl])
                plsc.subcore_barrier()

    return k(*h_blocks, srcb, dstb, ewb)


def _pool_kernel(y_blocks, batchb):
    K = len(y_blocks)
    db = 32
    chunks_per_tile = NP // CH // (NC * NS)
    prt = PG // NS

    @functools.partial(
        pl.kernel,
        out_type=tuple([jax.ShapeDtypeStruct((PG, db), F32)] * (2 * K)
                       + [jax.ShapeDtypeStruct((PG, LANES), F32)] * 2),
        mesh=_mesh(),
        compiler_params=_sc_params(),
        scratch_types=[pltpu.VMEM_SHARED((PG, db), F32)] * K + [
            pltpu.VMEM_SHARED((PG, LANES), F32),
            pltpu.VMEM((chunks_per_tile, CH), jnp.int32),
            pltpu.VMEM((CH, db), F32),
            pltpu.VMEM((CH, LANES), F32),
            pltpu.VMEM((prt, db), F32),
            pltpu.VMEM((prt, LANES), F32),
            pltpu.SemaphoreType.DMA,
        ],
    )
    def k(*refs):
        y_hbms = refs[:K]
        batch_hbm = refs[K]
        pouts = refs[K + 1:K + 1 + 2 * K]
        couts = refs[K + 1 + 2 * K:K + 3 + 2 * K]
        accs = refs[K + 3 + 2 * K:2 * K + 3 + 2 * K]
        cacc, bidx, rows, ones, zbuf, zbuf16, sem = refs[2 * K + 3 + 2 * K:]
        cid = lax.axis_index("c")
        sid = lax.axis_index("s")
        wid = cid * NS + sid
        _zero_zbuf(zbuf, db)
        _zero_zbuf(zbuf16, LANES)
        for a in accs:
            pltpu.sync_copy(zbuf, a.at[pl.ds(sid * prt, prt)])
        pltpu.sync_copy(zbuf16, cacc.at[pl.ds(sid * prt, prt)])

        @pl.loop(0, CH)
        def _(i):
            ones[i, pl.ds(0, LANES)] = jnp.ones((LANES,), F32)

        pltpu.sync_copy(batch_hbm.at[wid], bidx)
        plsc.subcore_barrier()

        @pl.loop(0, chunks_per_tile)
        def _(j):
            cb = wid * chunks_per_tile + j
            for blk in range(K):
                pltpu.sync_copy(y_hbms[blk].at[pl.ds(cb * CH, CH)], rows)
                pltpu.sync_copy(rows, accs[blk].at[bidx.at[j]], add=True)
            pltpu.sync_copy(ones, cacc.at[bidx.at[j]], add=True)

        plsc.subcore_barrier()
        sl = pl.ds(sid * prt, prt)

        @pl.when(cid == 0)
        def _():
            for blk in range(K):
                pltpu.sync_copy(accs[blk].at[sl], pouts[blk].at[sl])
            pltpu.sync_copy(cacc.at[sl], couts[0].at[sl])

        @pl.when(cid == 1)
        def _():
            for blk in range(K):
                pltpu.sync_copy(accs[blk].at[sl], pouts[K + blk].at[sl])
            pltpu.sync_copy(cacc.at[sl], couts[1].at[sl])

    return k(*y_blocks, batchb)


def _first_matmul(x_pad, w1p):
    def body(xr, wr, orf):
        orf[...] = jnp.dot(xr[...], wr[...], preferred_element_type=F32)

    return pl.pallas_call(
        body,
        grid=(NP // 1024,),
        in_specs=[
            pl.BlockSpec((1024, 128), lambda i: (i, 0)),
            pl.BlockSpec((128, 16), lambda i: (0, 0)),
        ],
        out_specs=pl.BlockSpec((1024, 16), lambda i: (i, 0)),
        out_shape=jax.ShapeDtypeStruct((NP, 16), F32),
    )(x_pad, w1p)


def _stats(parts):
    db = parts[0].shape[1]
    nparts = len(parts)

    def body(*refs):
        zb = refs[0][...]
        for r in refs[1:nparts]:
            zb = zb + r[...]
        orf = refs[nparts]
        s = jnp.sum(zb, axis=0)[None, :]
        q = jnp.sum(zb * zb, axis=0)[None, :]
        upd = jnp.concatenate([s, q, jnp.zeros((6, db), F32)], axis=0)

        @pl.when(pl.program_id(0) == 0)
        def _():
            orf[...] = jnp.zeros_like(orf)

        orf[...] += upd

    return pl.pallas_call(
        body,
        grid=(N // 1000,),
        in_specs=[pl.BlockSpec((1000, db), lambda i: (i, 0))] * nparts,
        out_specs=pl.BlockSpec((8, db), lambda i: (0, 0)),
        out_shape=jax.ShapeDtypeStruct((8, db), F32),
    )(*parts)


def _apply(part_groups, stats_list, g_list, be_list, wn_blocks=None,
           out_db=16):
    B = len(part_groups)
    db = part_groups[0][0].shape[1]
    npart = len(part_groups[0])
    if wn_blocks is not None:
        dnext = wn_blocks[0].shape[1]
        nout = dnext // out_db
        out_w = out_db
    else:
        nout = B
        out_w = db

    def body(*refs):
        i = 0
        zs = []
        for b in range(B):
            z = refs[i][...]
            for p in range(1, npart):
                z = z + refs[i + p][...]
            i += npart
            zs.append(z)
        sts = [refs[i + b][...] for b in range(B)]
        i += B
        gs = [refs[i + b][...] for b in range(B)]
        i += B
        bes = [refs[i + b][...] for b in range(B)]
        i += B
        wns = None
        if wn_blocks is not None:
            wns = [refs[i + b][...] for b in range(B)]
            i += B
        orefs = refs[i:]
        ys = []
        for b in range(B):
            m = sts[b][0:1, :] / float(N)
            v = sts[b][1:2, :] / float(N) - m * m
            r = lax.rsqrt(v + 1e-5)
            y = (zs[b] - m) * (r * gs[b]) + bes[b]
            ys.append(jnp.where(y >= 0, y, 0.01 * y))
        if wns is None:
            for b in range(B):
                orefs[b][...] = ys[b]
        else:
            acc = jnp.dot(ys[0], wns[0], preferred_element_type=F32)
            for b in range(1, B):
                acc += jnp.dot(ys[b], wns[b], preferred_element_type=F32)
            for j in range(nout):
                orefs[j][...] = acc[:, j * out_db:(j + 1) * out_db]

    in_specs = []
    args = []
    for b in range(B):
        for p in part_groups[b]:
            args.append(p)
            in_specs.append(pl.BlockSpec((1024, db), lambda i: (i, 0)))
    for st in stats_list:
        args.append(st)
        in_specs.append(pl.BlockSpec((8, db), lambda i: (0, 0)))
    for gl in g_list:
        args.append(gl)
        in_specs.append(pl.BlockSpec((1, db), lambda i: (0, 0)))
    for bl in be_list:
        args.append(bl)
        in_specs.append(pl.BlockSpec((1, db), lambda i: (0, 0)))
    if wn_blocks is not None:
        for w in wn_blocks:
            args.append(w)
            in_specs.append(pl.BlockSpec(w.shape, lambda i: (0, 0)))

    return pl.pallas_call(
        body,
        grid=(NP // 1024,),
        in_specs=in_specs,
        out_specs=[pl.BlockSpec((1024, out_w), lambda i: (i, 0))] * nout,
        out_shape=[jax.ShapeDtypeStruct((NP, out_w), F32)] * nout,
    )(*args)


def _fc(pool_parts, cnt_parts, wf1p, bf1p, wf2p, bf2p, wf3p, bf3p):
    K = 4

    def body(*refs):
        ps = refs[:2 * K]
        c0, c1 = refs[2 * K], refs[2 * K + 1]
        w1, b1, w2, b2, w3, b3 = refs[2 * K + 2:2 * K + 8]
        orf = refs[2 * K + 8]
        sums = [ps[b][...] + ps[K + b][...] for b in range(K)]
        s = jnp.concatenate(sums, axis=1)
        cnt = (c0[...] + c1[...])[:, 0:1]
        pooled = s / jnp.maximum(cnt, 1.0)
        a = jnp.dot(pooled, w1[...], preferred_element_type=F32) + b1[...]
        a = jnp.where(a >= 0, a, 0.01 * a)
        a = jnp.dot(a, w2[...], preferred_element_type=F32) + b2[...]
        a = jnp.where(a >= 0, a, 0.01 * a)
        orf[...] = jnp.dot(a, w3[...], preferred_element_type=F32) + b3[...]

    in_specs = (
        [pl.BlockSpec((1024, 32), lambda i: (0, 0))] * (2 * K)
        + [pl.BlockSpec((1024, 16), lambda i: (0, 0))] * 2
        + [pl.BlockSpec((128, 32), lambda i: (0, 0)),
           pl.BlockSpec((1, 32), lambda i: (0, 0)),
           pl.BlockSpec((32, 32), lambda i: (0, 0)),
           pl.BlockSpec((1, 32), lambda i: (0, 0)),
           pl.BlockSpec((32, 128), lambda i: (0, 0)),
           pl.BlockSpec((1, 128), lambda i: (0, 0))]
    )
    return pl.pallas_call(
        body,
        grid=(1,),
        in_specs=in_specs,
        out_specs=pl.BlockSpec((1024, 128), lambda i: (0, 0)),
        out_shape=jax.ShapeDtypeStruct((1024, 128), F32),
    )(*pool_parts, *cnt_parts, wf1p, bf1p, wf2p, bf2p, wf3p, bf3p)


def kernel(x, edge_index, edge_weight, batch, params):
    E = edge_weight.shape[0]
    pad_e = EP - E
    src = edge_index[0].astype(jnp.int32)
    dst = edge_index[1].astype(jnp.int32)
    pad_idx = (jnp.arange(pad_e, dtype=jnp.int32) * 131) % N
    srcb = jnp.concatenate([src, pad_idx]).reshape(EP // CH, CH)
    dstb = jnp.concatenate([dst, pad_idx]).reshape(EP // CH, CH)
    ewb = jnp.concatenate(
        [edge_weight.astype(F32), jnp.zeros((pad_e,), F32)]
    ).reshape(EP // CH, CH)
    batchb = jnp.concatenate(
        [batch.astype(jnp.int32), jnp.full((NP - N,), G, jnp.int32)]
    ).reshape(NC * NS, NP // CH // (NC * NS), CH)

    p = params
    x_pad = jnp.zeros((NP, 128), F32).at[:N, :6].set(x.astype(F32))
    w1p = jnp.zeros((128, 16), F32).at[:6].set(p["W1"])

    def row(a):
        return a.reshape(1, -1)

    h = _first_matmul(x_pad, w1p)
    p0, p1 = _scatter_layer_split_edges(16, h, srcb, dstb, ewb)
    st = _stats([p0, p1])
    (h,) = _apply([[p0, p1]], [st], [row(p["g1"])], [row(p["be1"])],
                  wn_blocks=[p["W2"]], out_db=16)
    p0, p1 = _scatter_layer_split_edges(16, h, srcb, dstb, ewb)
    st = _stats([p0, p1])
    (h,) = _apply([[p0, p1]], [st], [row(p["g2"])], [row(p["be2"])],
                  wn_blocks=[p["W3"]], out_db=32)
    p0, p1 = _scatter_layer_split_edges(32, h, srcb, dstb, ewb)
    st = _stats([p0, p1])
    h4 = _apply([[p0, p1]], [st], [row(p["g3"])], [row(p["be3"])],
                wn_blocks=[p["W4"]], out_db=32)
    z4 = _scatter_layer_split_cols(list(h4), srcb, dstb, ewb)
    st4 = [_stats([z]) for z in z4]
    g4 = [row(p["g4"][:32]), row(p["g4"][32:])]
    be4 = [row(p["be4"][:32]), row(p["be4"][32:])]
    w5b = [p["W5"][:32], p["W5"][32:]]
    h5 = _apply([[z] for z in z4], st4, g4, be4, wn_blocks=w5b, out_db=32)
    z5 = _scatter_layer_split_cols(list(h5), srcb, dstb, ewb)
    st5 = [_stats([z]) for z in z5]
    g5 = [row(p["g5"][32 * b:32 * (b + 1)]) for b in range(4)]
    be5 = [row(p["be5"][32 * b:32 * (b + 1)]) for b in range(4)]
    y5 = _apply([[z] for z in z5], st5, g5, be5, wn_blocks=None)
    pool = _pool_kernel(list(y5), batchb)
    pool_parts, cnt_parts = pool[:8], pool[8:]
    wf1p = jnp.zeros((128, 32), F32).at[:, :30].set(p["Wf1"])
    bf1p = jnp.zeros((1, 32), F32).at[0, :30].set(p["bf1"])
    wf2p = jnp.zeros((32, 32), F32).at[:30, :20].set(p["Wf2"])
    bf2p = jnp.zeros((1, 32), F32).at[0, :20].set(p["bf2"])
    wf3p = jnp.zeros((32, 128), F32).at[:20, :2].set(p["Wf3"])
    bf3p = jnp.zeros((1, 128), F32).at[0, :2].set(p["bf3"])
    out = _fc(pool_parts, cnt_parts, wf1p, bf1p, wf2p, bf2p, wf3p, bf3p)
    return out[:, :2]
